# Initial kernel scaffold; baseline (speedup 1.0000x reference)
import jax
import jax.numpy as jnp
from jax.experimental import pallas as pl


def kernel(x_country, x_product, edge_src, edge_dst, Wc, bc, Wp, bp, W1_cp_l, b1_cp, W1_cp_r, W1_pc_l, b1_pc, W1_pc_r, W2_cp_l, b2_cp, W2_cp_r, W2_pc_l, b2_pc, W2_pc_r, Wih_c, Whh_c, bih_c, bhh_c, Wih_p, Whh_p, bih_p, bhh_p):
    raise NotImplementedError("write your pallas kernel here")


# speedup vs baseline: 1.7119x; 1.7119x over previous
import functools

import jax
import jax.numpy as jnp
from jax import lax
from jax.experimental import pallas as pl
from jax.experimental.pallas import tpu as pltpu
from jax.experimental.pallas import tpu_sc as plsc

T = 4
NC_N = 10000
NP_N = 50000
DC = 64
DP = 32
H = 128
E = 320000

N_CORES = 2
N_SUB = 16
GROUP = 128
GK = 8
EPAD_SUB = 160 * GROUP
E_PAD = N_SUB * EPAD_SUB
N_GROUPS = E_PAD // GROUP
ZCH = 64

f32 = jnp.float32
i32 = jnp.int32


def _ceil_to(x, m):
  return (x + m - 1) // m * m


def _encode_sl_body(x_ref, w_ref, b_ref, o_ref):
  h = jnp.dot(x_ref[0], w_ref[...], preferred_element_type=f32) + b_ref[...]
  for j in range(4):
    o_ref[j] = h[:, 32 * j:32 * (j + 1)]


def _encode_st_body(x_ref, w_ref, b_ref, o_ref):
  o_ref[0] = jnp.dot(x_ref[0], w_ref[...], preferred_element_type=f32) + b_ref[...]


def _encode(x, w, b, n, d, bsz, slice_major):
  body = _encode_sl_body if slice_major else _encode_st_body
  nsl = 16 if slice_major else T
  oblk = (4, bsz, 32) if slice_major else (1, bsz, H)
  return pl.pallas_call(
      body,
      grid=(T, n // bsz),
      in_specs=[
          pl.BlockSpec((1, bsz, d), lambda t, i: (t, i, 0)),
          pl.BlockSpec((d, H), lambda t, i: (0, 0)),
          pl.BlockSpec((1, H), lambda t, i: (0, 0)),
      ],
      out_specs=pl.BlockSpec(oblk, lambda t, i: (t, i, 0)),
      out_shape=jax.ShapeDtypeStruct((nsl, n, oblk[2]), f32),
  )(x, w, b)


def _combine_body(relu, seg_sl, dst_sl, out_sl,
                  seg_ref, cnt_ref, xd_ref, wl_ref, bl_ref, wr_ref, o_ref):
  if seg_sl:
    seg = jnp.concatenate([seg_ref[j] for j in range(4)], axis=1)
  else:
    seg = seg_ref[0]
  if dst_sl:
    xd = jnp.concatenate([xd_ref[j] for j in range(4)], axis=1)
  else:
    xd = xd_ref[0]
  cnt = jnp.maximum(cnt_ref[:, 0:1], 1.0)
  mean = seg / cnt
  h = (jnp.dot(mean, wl_ref[...], preferred_element_type=f32) + bl_ref[...]
       + jnp.dot(xd, wr_ref[...], preferred_element_type=f32))
  if relu:
    h = jnp.maximum(h, 0.0)
  if out_sl:
    for j in range(4):
      o_ref[j] = h[:, 32 * j:32 * (j + 1)]
  else:
    o_ref[0] = h


def _combine(seg, cnt, xd, wl, bl, wr, n, bsz, relu, out_sl):
  seg_sl = seg.shape[0] == 16
  dst_sl = xd.shape[0] == 16
  sblk = (4, bsz, 32) if seg_sl else (1, bsz, H)
  dblk = (4, bsz, 32) if dst_sl else (1, bsz, H)
  oblk = (4, bsz, 32) if out_sl else (1, bsz, H)
  nsl = 16 if out_sl else T
  body = functools.partial(_combine_body, relu, seg_sl, dst_sl, out_sl)
  return pl.pallas_call(
      body,
      grid=(T, n // bsz),
      in_specs=[
          pl.BlockSpec(sblk, lambda t, i: (t, i, 0)),
          pl.BlockSpec((bsz, 16), lambda t, i: (i, 0)),
          pl.BlockSpec(dblk, lambda t, i: (t, i, 0)),
          pl.BlockSpec((H, H), lambda t, i: (0, 0)),
          pl.BlockSpec((1, H), lambda t, i: (0, 0)),
          pl.BlockSpec((H, H), lambda t, i: (0, 0)),
      ],
      out_specs=pl.BlockSpec(oblk, lambda t, i: (t, i, 0)),
      out_shape=jax.ShapeDtypeStruct((nsl, n, oblk[2]), f32),
  )(seg, cnt, xd, wl, bl, wr)


def _gru_body(seq_ref, wih_ref, whh_ref, bih_ref, bhh_ref, o_ref):
  b = seq_ref.shape[1]
  h = jnp.zeros((b, H), f32)
  for t in range(T):
    x = seq_ref[t]
    gi = jnp.dot(x, wih_ref[...], preferred_element_type=f32) + bih_ref[...]
    gh = jnp.dot(h, whh_ref[...], preferred_element_type=f32) + bhh_ref[...]
    r = jax.nn.sigmoid(gi[:, 0:H] + gh[:, 0:H])
    z = jax.nn.sigmoid(gi[:, H:2 * H] + gh[:, H:2 * H])
    nn = jnp.tanh(gi[:, 2 * H:] + r * gh[:, 2 * H:])
    h = (1.0 - z) * nn + z * h
  o_ref[...] = h


def _gru(seq, wih, whh, bih, bhh, n, bsz):
  return pl.pallas_call(
      _gru_body,
      grid=(n // bsz,),
      in_specs=[
          pl.BlockSpec((T, bsz, H), lambda i: (0, i, 0)),
          pl.BlockSpec((H, 3 * H), lambda i: (0, 0)),
          pl.BlockSpec((H, 3 * H), lambda i: (0, 0)),
          pl.BlockSpec((1, 3 * H), lambda i: (0, 0)),
          pl.BlockSpec((1, 3 * H), lambda i: (0, 0)),
      ],
      out_specs=pl.BlockSpec((bsz, H), lambda i: (i, 0)),
      out_shape=jax.ShapeDtypeStruct((n, H), f32),
  )(seq, wih, whh, bih, bhh)


def _fill(buf, val, nrows, sw):
  v = jnp.full((16,), val, f32)
  def row(r, _):
    for k in range(sw // 16):
      buf[r, pl.ds(16 * k, 16)] = v
    return 0
  lax.fori_loop(0, nrows, row, 0)


def _spmm_sc(x, srcg, dstg, n_src, n_dst, sw):
  nsl = 512 // sw
  passes = nsl // N_CORES
  n_out = _ceil_to(n_dst, GROUP)
  n_dst_pad = _ceil_to(n_out + 1, N_SUB * ZCH)
  r0 = n_dst_pad // N_SUB
  w0 = n_out // N_SUB
  mesh = plsc.VectorSubcoreMesh(core_axis_name="c", subcore_axis_name="s")

  @functools.partial(
      pl.kernel,
      out_type=jax.ShapeDtypeStruct((nsl, n_out, sw), f32),
      mesh=mesh,
      scratch_types=[
          pltpu.VMEM_SHARED((n_dst_pad, sw), f32),
          pltpu.VMEM((ZCH, sw), f32),
          pltpu.VMEM((GK, GROUP), i32),
          pltpu.VMEM((GK, GROUP), i32),
          pltpu.VMEM((GROUP, sw), f32),
          pltpu.SemaphoreType.DMA,
      ],
      compiler_params=pltpu.CompilerParams(use_tc_tiling_on_sc=False),
  )
  def run(x_hbm, src_hbm, dst_hbm, out_hbm, acc, zbuf, sidx, didx, rows, sem):
    c = lax.axis_index("c")
    s = lax.axis_index("s")
    _fill(zbuf, 0.0, ZCH, sw)
    for p in range(passes):
      sl = c * passes + p
      def zero_step(k, _):
        pltpu.sync_copy(zbuf, acc.at[pl.ds(s * r0 + k * ZCH, ZCH)])
        return 0
      lax.fori_loop(0, r0 // ZCH, zero_step, 0)
      plsc.subcore_barrier()

      def edge_step(g, _):
        grow = s * (EPAD_SUB // GROUP) + g * GK
        pltpu.sync_copy(src_hbm.at[pl.ds(grow, GK)], sidx)
        pltpu.sync_copy(dst_hbm.at[pl.ds(grow, GK)], didx)
        for j in range(GK):
          pltpu.async_copy(x_hbm.at[sl].at[sidx.at[j]], rows, sem).wait()
          pltpu.sync_copy(rows, acc.at[didx.at[j]], add=True)
        return 0
      lax.fori_loop(0, EPAD_SUB // GROUP // GK, edge_step, 0)
      plsc.subcore_barrier()
      pltpu.sync_copy(acc.at[pl.ds(s * w0, w0)],
                      out_hbm.at[sl].at[pl.ds(s * w0, w0)])
      plsc.subcore_barrier()

  return run(x, srcg, dstg)


def _counts_sc(cpg, pcg):
  np_out = _ceil_to(NP_N, GROUP)
  nc_out = _ceil_to(NC_N, GROUP)
  np_pad = _ceil_to(np_out + 1, N_SUB * ZCH)
  nc_pad = _ceil_to(nc_out + 1, N_SUB * ZCH)
  mesh = plsc.VectorSubcoreMesh(core_axis_name="c", subcore_axis_name="s")

  @functools.partial(
      pl.kernel,
      out_type=(jax.ShapeDtypeStruct((np_out, 16), f32),
                jax.ShapeDtypeStruct((nc_out, 16), f32)),
      mesh=mesh,
      scratch_types=[
          pltpu.VMEM_SHARED((np_pad, 16), f32),
          pltpu.VMEM_SHARED((nc_pad, 16), f32),
          pltpu.VMEM((ZCH, 16), f32),
          pltpu.VMEM((GK, GROUP), i32),
          pltpu.VMEM((GROUP, 16), f32),
      ],
      compiler_params=pltpu.CompilerParams(use_tc_tiling_on_sc=False),
  )
  def run(cp_hbm, pc_hbm, outp_hbm, outc_hbm, accp, accc, zbuf, didx, ones):
    c = lax.axis_index("c")
    s = lax.axis_index("s")
    _fill(zbuf, 0.0, ZCH, 16)
    _fill(ones, 1.0, GROUP, 16)

    for core, acc, idx_hbm, out_hbm, n_out, n_pad in (
        (0, accp, cp_hbm, outp_hbm, np_out, np_pad),
        (1, accc, pc_hbm, outc_hbm, nc_out, nc_pad),
    ):
      r0 = n_pad // N_SUB
      w0 = n_out // N_SUB

      @pl.when(c == core)
      def _():
        def zero_step(k, _):
          pltpu.sync_copy(zbuf, acc.at[pl.ds(s * r0 + k * ZCH, ZCH)])
          return 0
        lax.fori_loop(0, r0 // ZCH, zero_step, 0)
        plsc.subcore_barrier()

        def edge_step(g, _):
          grow = s * (EPAD_SUB // GROUP) + g * GK
          pltpu.sync_copy(idx_hbm.at[pl.ds(grow, GK)], didx)
          for j in range(GK):
            pltpu.sync_copy(ones, acc.at[didx.at[j]], add=True)
          return 0
        lax.fori_loop(0, EPAD_SUB // GROUP // GK, edge_step, 0)
        plsc.subcore_barrier()
        pltpu.sync_copy(acc.at[pl.ds(s * w0, w0)], out_hbm.at[pl.ds(s * w0, w0)])

  return run(cpg, pcg)


def kernel(x_country, x_product, edge_src, edge_dst, Wc, bc, Wp, bp,
           W1_cp_l, b1_cp, W1_cp_r, W1_pc_l, b1_pc, W1_pc_r,
           W2_cp_l, b2_cp, W2_cp_r, W2_pc_l, b2_pc, W2_pc_r,
           Wih_c, Whh_c, bih_c, bhh_c, Wih_p, Whh_p, bih_p, bhh_p):
  es = edge_src.astype(i32)
  ed = edge_dst.astype(i32)
  pad = E_PAD - E

  def _g(a, fill):
    return jnp.concatenate([a, jnp.full((pad,), fill, i32)]).reshape(
        N_GROUPS, GROUP)

  cp_src = _g(es, 0)
  cp_dst = _g(ed, NP_N)
  pc_src = _g(ed, 0)
  pc_dst = _g(es, NC_N)

  r1 = lambda b: b.reshape(1, -1)

  cnt_p, cnt_c = _counts_sc(cp_dst, pc_dst)

  hc = _encode(x_country, Wc, r1(bc), NC_N, DC, 2000, slice_major=True)
  hp = _encode(x_product, Wp, r1(bp), NP_N, DP, 2000, slice_major=False)

  seg_p1 = _spmm_sc(hc, cp_src, cp_dst, NC_N, NP_N, 32)
  seg_c1 = _spmm_sc(hp, pc_src, pc_dst, NP_N, NC_N, 128)

  p1 = _combine(seg_p1, cnt_p, hp, W1_cp_l, r1(b1_cp), W1_cp_r,
                NP_N, 2000, relu=True, out_sl=False)
  c1 = _combine(seg_c1, cnt_c, hc, W1_pc_l, r1(b1_pc), W1_pc_r,
                NC_N, 2000, relu=True, out_sl=True)

  seg_p2 = _spmm_sc(c1, cp_src, cp_dst, NC_N, NP_N, 32)
  seg_c2 = _spmm_sc(p1, pc_src, pc_dst, NP_N, NC_N, 128)

  p2 = _combine(seg_p2, cnt_p, p1, W2_cp_l, r1(b2_cp), W2_cp_r,
                NP_N, 2000, relu=False, out_sl=False)
  c2 = _combine(seg_c2, cnt_c, c1, W2_pc_l, r1(b2_pc), W2_pc_r,
                NC_N, 2000, relu=False, out_sl=False)

  zc = _gru(c2, Wih_c, Whh_c, r1(bih_c), r1(bhh_c), NC_N, 2000)
  zp = _gru(p2, Wih_p, Whh_p, r1(bih_p), r1(bhh_p), NP_N, 2000)
  return (zc, zp)


# speedup vs baseline: 2.0072x; 1.1725x over previous
import functools

import jax
import jax.numpy as jnp
from jax import lax
from jax.experimental import pallas as pl
from jax.experimental.pallas import tpu as pltpu
from jax.experimental.pallas import tpu_sc as plsc

T = 4
NC_N = 10000
NP_N = 50000
DC = 64
DP = 32
H = 128
E = 320000

N_CORES = 2
N_SUB = 16
GROUP = 128
GK = 8
EPAD_SUB = 160 * GROUP
E_PAD = N_SUB * EPAD_SUB
N_GROUPS = E_PAD // GROUP
ZCH = 32

f32 = jnp.float32
i32 = jnp.int32


def _ceil_to(x, m):
  return (x + m - 1) // m * m


def _encode_sl_body(x_ref, w_ref, b_ref, o_ref):
  h = jnp.dot(x_ref[0], w_ref[...], preferred_element_type=f32) + b_ref[...]
  for j in range(4):
    o_ref[j] = h[:, 32 * j:32 * (j + 1)]


def _encode_st_body(x_ref, w_ref, b_ref, o_ref):
  o_ref[0] = jnp.dot(x_ref[0], w_ref[...], preferred_element_type=f32) + b_ref[...]


def _encode(x, w, b, n, d, bsz, slice_major):
  body = _encode_sl_body if slice_major else _encode_st_body
  nsl = 16 if slice_major else T
  oblk = (4, bsz, 32) if slice_major else (1, bsz, H)
  return pl.pallas_call(
      body,
      grid=(T, n // bsz),
      in_specs=[
          pl.BlockSpec((1, bsz, d), lambda t, i: (t, i, 0)),
          pl.BlockSpec((d, H), lambda t, i: (0, 0)),
          pl.BlockSpec((1, H), lambda t, i: (0, 0)),
      ],
      out_specs=pl.BlockSpec(oblk, lambda t, i: (t, i, 0)),
      out_shape=jax.ShapeDtypeStruct((nsl, n, oblk[2]), f32),
  )(x, w, b)


def _combine_body(relu, seg_sl, dst_sl, out_sl,
                  seg_ref, cnt_ref, xd_ref, wl_ref, bl_ref, wr_ref, o_ref):
  if seg_sl:
    seg = jnp.concatenate([seg_ref[j] for j in range(4)], axis=1)
  else:
    seg = seg_ref[0]
  if dst_sl:
    xd = jnp.concatenate([xd_ref[j] for j in range(4)], axis=1)
  else:
    xd = xd_ref[0]
  cnt = jnp.maximum(cnt_ref[:, 0:1], 1.0)
  mean = seg / cnt
  h = (jnp.dot(mean, wl_ref[...], preferred_element_type=f32) + bl_ref[...]
       + jnp.dot(xd, wr_ref[...], preferred_element_type=f32))
  if relu:
    h = jnp.maximum(h, 0.0)
  if out_sl:
    for j in range(4):
      o_ref[j] = h[:, 32 * j:32 * (j + 1)]
  else:
    o_ref[0] = h


def _combine(seg, cnt, xd, wl, bl, wr, n, bsz, relu, out_sl):
  seg_sl = seg.shape[0] == 16
  dst_sl = xd.shape[0] == 16
  sblk = (4, bsz, 32) if seg_sl else (1, bsz, H)
  dblk = (4, bsz, 32) if dst_sl else (1, bsz, H)
  oblk = (4, bsz, 32) if out_sl else (1, bsz, H)
  nsl = 16 if out_sl else T
  body = functools.partial(_combine_body, relu, seg_sl, dst_sl, out_sl)
  return pl.pallas_call(
      body,
      grid=(T, n // bsz),
      in_specs=[
          pl.BlockSpec(sblk, lambda t, i: (t, i, 0)),
          pl.BlockSpec((bsz, 16), lambda t, i: (i, 0)),
          pl.BlockSpec(dblk, lambda t, i: (t, i, 0)),
          pl.BlockSpec((H, H), lambda t, i: (0, 0)),
          pl.BlockSpec((1, H), lambda t, i: (0, 0)),
          pl.BlockSpec((H, H), lambda t, i: (0, 0)),
      ],
      out_specs=pl.BlockSpec(oblk, lambda t, i: (t, i, 0)),
      out_shape=jax.ShapeDtypeStruct((nsl, n, oblk[2]), f32),
  )(seg, cnt, xd, wl, bl, wr)


def _gru_body(seq_ref, wih_ref, whh_ref, bih_ref, bhh_ref, o_ref):
  b = seq_ref.shape[1]
  h = jnp.zeros((b, H), f32)
  for t in range(T):
    x = seq_ref[t]
    gi = jnp.dot(x, wih_ref[...], preferred_element_type=f32) + bih_ref[...]
    gh = jnp.dot(h, whh_ref[...], preferred_element_type=f32) + bhh_ref[...]
    r = jax.nn.sigmoid(gi[:, 0:H] + gh[:, 0:H])
    z = jax.nn.sigmoid(gi[:, H:2 * H] + gh[:, H:2 * H])
    nn = jnp.tanh(gi[:, 2 * H:] + r * gh[:, 2 * H:])
    h = (1.0 - z) * nn + z * h
  o_ref[...] = h


def _gru(seq, wih, whh, bih, bhh, n, bsz):
  return pl.pallas_call(
      _gru_body,
      grid=(n // bsz,),
      in_specs=[
          pl.BlockSpec((T, bsz, H), lambda i: (0, i, 0)),
          pl.BlockSpec((H, 3 * H), lambda i: (0, 0)),
          pl.BlockSpec((H, 3 * H), lambda i: (0, 0)),
          pl.BlockSpec((1, 3 * H), lambda i: (0, 0)),
          pl.BlockSpec((1, 3 * H), lambda i: (0, 0)),
      ],
      out_specs=pl.BlockSpec((bsz, H), lambda i: (i, 0)),
      out_shape=jax.ShapeDtypeStruct((n, H), f32),
  )(seq, wih, whh, bih, bhh)


def _fill(buf, val, nrows, sw):
  v = jnp.full((16,), val, f32)
  def row(r, _):
    for k in range(sw // 16):
      buf[r, pl.ds(16 * k, 16)] = v
    return 0
  lax.fori_loop(0, nrows, row, 0)


def _spmm_sc(x, srcg, dstg, n_src, n_dst, sw):
  nsl = 512 // sw
  passes = nsl // N_CORES
  n_out = _ceil_to(n_dst, GROUP)
  n_dst_pad = _ceil_to(n_out + 1, N_SUB * ZCH)
  r0 = n_dst_pad // N_SUB
  w0 = n_out // N_SUB
  mesh = plsc.VectorSubcoreMesh(core_axis_name="c", subcore_axis_name="s")

  @functools.partial(
      pl.kernel,
      out_type=jax.ShapeDtypeStruct((nsl, n_out, sw), f32),
      mesh=mesh,
      scratch_types=[
          pltpu.VMEM_SHARED((n_dst_pad, sw), f32),
          pltpu.VMEM((ZCH, sw), f32),
          pltpu.VMEM((GK, GROUP), i32),
          pltpu.VMEM((GK, GROUP), i32),
          pltpu.VMEM((2, 2 if sw <= 32 else 1, GROUP, sw), f32),
          pltpu.SemaphoreType.DMA,
          pltpu.SemaphoreType.DMA,
      ],
      compiler_params=pltpu.CompilerParams(use_tc_tiling_on_sc=False),
  )
  def run(x_hbm, src_hbm, dst_hbm, out_hbm, acc, zbuf, sidx, didx, rows, sem,
          ssem):
    c = lax.axis_index("c")
    s = lax.axis_index("s")
    _fill(zbuf, 0.0, ZCH, sw)
    for p in range(passes):
      sl = c * passes + p
      def zero_step(k, _):
        pltpu.sync_copy(zbuf, acc.at[pl.ds(s * r0 + k * ZCH, ZCH)])
        return 0
      lax.fori_loop(0, r0 // ZCH, zero_step, 0)
      plsc.subcore_barrier()

      nbw = 2 if sw <= 32 else 1
      waves = GK // nbw

      def edge_step(g, _):
        grow = s * (EPAD_SUB // GROUP) + g * GK
        pltpu.sync_copy(src_hbm.at[pl.ds(grow, GK)], sidx)
        pltpu.sync_copy(dst_hbm.at[pl.ds(grow, GK)], didx)
        scat = [[], []]
        for w in range(waves):
          st = w % 2
          for d in scat[st]:
            d.wait()
          scat[st] = []
          gd = []
          for k in range(nbw):
            j = w * nbw + k
            gd.append(pltpu.async_copy(
                x_hbm.at[sl].at[sidx.at[j]], rows.at[st, k], sem))
          for d in gd:
            d.wait()
          for k in range(nbw):
            j = w * nbw + k
            scat[st].append(pltpu.async_copy(
                rows.at[st, k], acc.at[didx.at[j]], ssem, add=True))
        for st in (0, 1):
          for d in scat[st]:
            d.wait()
        return 0
      lax.fori_loop(0, EPAD_SUB // GROUP // GK, edge_step, 0)
      plsc.subcore_barrier()
      pltpu.sync_copy(acc.at[pl.ds(s * w0, w0)],
                      out_hbm.at[sl].at[pl.ds(s * w0, w0)])
      plsc.subcore_barrier()

  return run(x, srcg, dstg)


def _counts_sc(cpg, pcg):
  np_out = _ceil_to(NP_N, GROUP)
  nc_out = _ceil_to(NC_N, GROUP)
  np_pad = _ceil_to(np_out + 1, N_SUB * ZCH)
  nc_pad = _ceil_to(nc_out + 1, N_SUB * ZCH)
  mesh = plsc.VectorSubcoreMesh(core_axis_name="c", subcore_axis_name="s")

  @functools.partial(
      pl.kernel,
      out_type=(jax.ShapeDtypeStruct((np_out, 16), f32),
                jax.ShapeDtypeStruct((nc_out, 16), f32)),
      mesh=mesh,
      scratch_types=[
          pltpu.VMEM_SHARED((np_pad, 16), f32),
          pltpu.VMEM_SHARED((nc_pad, 16), f32),
          pltpu.VMEM((ZCH, 16), f32),
          pltpu.VMEM((GK, GROUP), i32),
          pltpu.VMEM((GROUP, 16), f32),
      ],
      compiler_params=pltpu.CompilerParams(use_tc_tiling_on_sc=False),
  )
  def run(cp_hbm, pc_hbm, outp_hbm, outc_hbm, accp, accc, zbuf, didx, ones):
    c = lax.axis_index("c")
    s = lax.axis_index("s")
    _fill(zbuf, 0.0, ZCH, 16)
    _fill(ones, 1.0, GROUP, 16)

    for core, acc, idx_hbm, out_hbm, n_out, n_pad in (
        (0, accp, cp_hbm, outp_hbm, np_out, np_pad),
        (1, accc, pc_hbm, outc_hbm, nc_out, nc_pad),
    ):
      r0 = n_pad // N_SUB
      w0 = n_out // N_SUB

      @pl.when(c == core)
      def _():
        def zero_step(k, _):
          pltpu.sync_copy(zbuf, acc.at[pl.ds(s * r0 + k * ZCH, ZCH)])
          return 0
        lax.fori_loop(0, r0 // ZCH, zero_step, 0)
        plsc.subcore_barrier()

        def edge_step(g, _):
          grow = s * (EPAD_SUB // GROUP) + g * GK
          pltpu.sync_copy(idx_hbm.at[pl.ds(grow, GK)], didx)
          for j in range(GK):
            pltpu.sync_copy(ones, acc.at[didx.at[j]], add=True)
          return 0
        lax.fori_loop(0, EPAD_SUB // GROUP // GK, edge_step, 0)
        plsc.subcore_barrier()
        pltpu.sync_copy(acc.at[pl.ds(s * w0, w0)], out_hbm.at[pl.ds(s * w0, w0)])

  return run(cpg, pcg)


def kernel(x_country, x_product, edge_src, edge_dst, Wc, bc, Wp, bp,
           W1_cp_l, b1_cp, W1_cp_r, W1_pc_l, b1_pc, W1_pc_r,
           W2_cp_l, b2_cp, W2_cp_r, W2_pc_l, b2_pc, W2_pc_r,
           Wih_c, Whh_c, bih_c, bhh_c, Wih_p, Whh_p, bih_p, bhh_p):
  es = edge_src.astype(i32)
  ed = edge_dst.astype(i32)
  pad = E_PAD - E

  def _g(a, fill):
    return jnp.concatenate([a, jnp.full((pad,), fill, i32)]).reshape(
        N_GROUPS, GROUP)

  cp_src = _g(es, 0)
  cp_dst = _g(ed, NP_N)
  pc_src = _g(ed, 0)
  pc_dst = _g(es, NC_N)

  r1 = lambda b: b.reshape(1, -1)

  cnt_p, cnt_c = _counts_sc(cp_dst, pc_dst)

  hc = _encode(x_country, Wc, r1(bc), NC_N, DC, 2000, slice_major=True)
  hp = _encode(x_product, Wp, r1(bp), NP_N, DP, 2000, slice_major=False)

  seg_p1 = _spmm_sc(hc, cp_src, cp_dst, NC_N, NP_N, 32)
  seg_c1 = _spmm_sc(hp, pc_src, pc_dst, NP_N, NC_N, 128)

  p1 = _combine(seg_p1, cnt_p, hp, W1_cp_l, r1(b1_cp), W1_cp_r,
                NP_N, 2000, relu=True, out_sl=False)
  c1 = _combine(seg_c1, cnt_c, hc, W1_pc_l, r1(b1_pc), W1_pc_r,
                NC_N, 2000, relu=True, out_sl=True)

  seg_p2 = _spmm_sc(c1, cp_src, cp_dst, NC_N, NP_N, 32)
  seg_c2 = _spmm_sc(p1, pc_src, pc_dst, NP_N, NC_N, 128)

  p2 = _combine(seg_p2, cnt_p, p1, W2_cp_l, r1(b2_cp), W2_cp_r,
                NP_N, 2000, relu=False, out_sl=False)
  c2 = _combine(seg_c2, cnt_c, c1, W2_pc_l, r1(b2_pc), W2_pc_r,
                NC_N, 2000, relu=False, out_sl=False)

  zc = _gru(c2, Wih_c, Whh_c, r1(bih_c), r1(bhh_c), NC_N, 2000)
  zp = _gru(p2, Wih_p, Whh_p, r1(bih_p), r1(bhh_p), NP_N, 2000)
  return (zc, zp)


# speedup vs baseline: 2.6020x; 1.2964x over previous
import functools

import jax
import jax.numpy as jnp
from jax import lax
from jax.experimental import pallas as pl
from jax.experimental.pallas import tpu as pltpu
from jax.experimental.pallas import tpu_sc as plsc

T = 4
NC_N = 10000
NP_N = 50000
DC = 64
DP = 32
H = 128
E = 320000

N_CORES = 2
N_SUB = 16
GROUP = 128
GK = 8
EPAD_SUB = 160 * GROUP
E_PAD = N_SUB * EPAD_SUB
N_GROUPS = E_PAD // GROUP
ZCH = 32

f32 = jnp.float32
i32 = jnp.int32


def _ceil_to(x, m):
  return (x + m - 1) // m * m


def _encode_sl_body(x_ref, w_ref, b_ref, o_ref):
  h = jnp.dot(x_ref[0], w_ref[...], preferred_element_type=f32) + b_ref[...]
  for j in range(4):
    o_ref[j] = h[:, 32 * j:32 * (j + 1)]


def _encode_st_body(x_ref, w_ref, b_ref, o_ref):
  o_ref[0] = jnp.dot(x_ref[0], w_ref[...], preferred_element_type=f32) + b_ref[...]


def _encode(x, w, b, n, d, bsz, slice_major):
  body = _encode_sl_body if slice_major else _encode_st_body
  nsl = 16 if slice_major else T
  oblk = (4, bsz, 32) if slice_major else (1, bsz, H)
  return pl.pallas_call(
      body,
      grid=(T, n // bsz),
      in_specs=[
          pl.BlockSpec((1, bsz, d), lambda t, i: (t, i, 0)),
          pl.BlockSpec((d, H), lambda t, i: (0, 0)),
          pl.BlockSpec((1, H), lambda t, i: (0, 0)),
      ],
      out_specs=pl.BlockSpec(oblk, lambda t, i: (t, i, 0)),
      out_shape=jax.ShapeDtypeStruct((nsl, n, oblk[2]), f32),
  )(x, w, b)


def _combine_body(relu, seg_sl, dst_sl, out_sl,
                  seg_ref, cnt_ref, xd_ref, wl_ref, bl_ref, wr_ref, o_ref):
  if seg_sl:
    seg = jnp.concatenate([seg_ref[j] for j in range(4)], axis=1)
  else:
    seg = seg_ref[0]
  if dst_sl:
    xd = jnp.concatenate([xd_ref[j] for j in range(4)], axis=1)
  else:
    xd = xd_ref[0]
  cnt = jnp.maximum(cnt_ref[:, 0:1], 1.0)
  mean = seg / cnt
  h = (jnp.dot(mean, wl_ref[...], preferred_element_type=f32) + bl_ref[...]
       + jnp.dot(xd, wr_ref[...], preferred_element_type=f32))
  if relu:
    h = jnp.maximum(h, 0.0)
  if out_sl:
    for j in range(4):
      o_ref[j] = h[:, 32 * j:32 * (j + 1)]
  else:
    o_ref[0] = h


def _combine(seg, cnt, xd, wl, bl, wr, n, bsz, relu, out_sl):
  seg_sl = seg.shape[0] == 16
  dst_sl = xd.shape[0] == 16
  sblk = (4, bsz, 32) if seg_sl else (1, bsz, H)
  dblk = (4, bsz, 32) if dst_sl else (1, bsz, H)
  oblk = (4, bsz, 32) if out_sl else (1, bsz, H)
  nsl = 16 if out_sl else T
  body = functools.partial(_combine_body, relu, seg_sl, dst_sl, out_sl)
  return pl.pallas_call(
      body,
      grid=(T, n // bsz),
      in_specs=[
          pl.BlockSpec(sblk, lambda t, i: (t, i, 0)),
          pl.BlockSpec((bsz, 16), lambda t, i: (i, 0)),
          pl.BlockSpec(dblk, lambda t, i: (t, i, 0)),
          pl.BlockSpec((H, H), lambda t, i: (0, 0)),
          pl.BlockSpec((1, H), lambda t, i: (0, 0)),
          pl.BlockSpec((H, H), lambda t, i: (0, 0)),
      ],
      out_specs=pl.BlockSpec(oblk, lambda t, i: (t, i, 0)),
      out_shape=jax.ShapeDtypeStruct((nsl, n, oblk[2]), f32),
  )(seg, cnt, xd, wl, bl, wr)


def _gru_body(seq_ref, wih_ref, whh_ref, bih_ref, bhh_ref, o_ref):
  b = seq_ref.shape[1]
  h = jnp.zeros((b, H), f32)
  for t in range(T):
    x = seq_ref[t]
    gi = jnp.dot(x, wih_ref[...], preferred_element_type=f32) + bih_ref[...]
    gh = jnp.dot(h, whh_ref[...], preferred_element_type=f32) + bhh_ref[...]
    r = jax.nn.sigmoid(gi[:, 0:H] + gh[:, 0:H])
    z = jax.nn.sigmoid(gi[:, H:2 * H] + gh[:, H:2 * H])
    nn = jnp.tanh(gi[:, 2 * H:] + r * gh[:, 2 * H:])
    h = (1.0 - z) * nn + z * h
  o_ref[...] = h


def _gru(seq, wih, whh, bih, bhh, n, bsz):
  return pl.pallas_call(
      _gru_body,
      grid=(n // bsz,),
      in_specs=[
          pl.BlockSpec((T, bsz, H), lambda i: (0, i, 0)),
          pl.BlockSpec((H, 3 * H), lambda i: (0, 0)),
          pl.BlockSpec((H, 3 * H), lambda i: (0, 0)),
          pl.BlockSpec((1, 3 * H), lambda i: (0, 0)),
          pl.BlockSpec((1, 3 * H), lambda i: (0, 0)),
      ],
      out_specs=pl.BlockSpec((bsz, H), lambda i: (i, 0)),
      out_shape=jax.ShapeDtypeStruct((n, H), f32),
  )(seq, wih, whh, bih, bhh)


def _fill(buf, val, nrows, sw):
  v = jnp.full((16,), val, f32)
  def row(r, _):
    for k in range(sw // 16):
      buf[r, pl.ds(16 * k, 16)] = v
    return 0
  lax.fori_loop(0, nrows, row, 0)


def _spmm_sc(x, srcg, dstg, n_src, n_dst, sw):
  nsl = 512 // sw
  passes = nsl // N_CORES
  local_src = n_src * sw <= 400_000
  gk = 4 if local_src else GK
  n_out = _ceil_to(n_dst, GROUP)
  n_dst_pad = _ceil_to(n_out + 1, N_SUB * ZCH)
  r0 = n_dst_pad // N_SUB
  w0 = n_out // N_SUB
  t0 = n_src // N_SUB
  mesh = plsc.VectorSubcoreMesh(core_axis_name="c", subcore_axis_name="s")

  scratch = [
      pltpu.VMEM_SHARED((n_dst_pad, sw), f32),
      pltpu.VMEM((ZCH, sw), f32),
      pltpu.VMEM((gk, GROUP), i32),
      pltpu.VMEM((gk, GROUP), i32),
      pltpu.VMEM((2, 2 if sw <= 32 and not local_src else 1, GROUP, sw), f32),
      pltpu.SemaphoreType.DMA,
      pltpu.SemaphoreType.DMA,
  ]
  if local_src:
    scratch.append(pltpu.VMEM_SHARED((n_src, sw), f32))

  @functools.partial(
      pl.kernel,
      out_type=jax.ShapeDtypeStruct((nsl, n_out, sw), f32),
      mesh=mesh,
      scratch_types=scratch,
      compiler_params=pltpu.CompilerParams(use_tc_tiling_on_sc=False),
  )
  def run(x_hbm, src_hbm, dst_hbm, out_hbm, acc, zbuf, sidx, didx, rows, sem,
          ssem, *tbl):
    c = lax.axis_index("c")
    s = lax.axis_index("s")
    _fill(zbuf, 0.0, ZCH, sw)
    for p in range(passes):
      sl = c * passes + p
      def zero_step(k, _):
        pltpu.sync_copy(zbuf, acc.at[pl.ds(s * r0 + k * ZCH, ZCH)])
        return 0
      lax.fori_loop(0, r0 // ZCH, zero_step, 0)
      if local_src:
        pltpu.sync_copy(x_hbm.at[sl].at[pl.ds(s * t0, t0)],
                        tbl[0].at[pl.ds(s * t0, t0)])
      plsc.subcore_barrier()
      src_tab = tbl[0] if local_src else x_hbm.at[sl]

      nbw = 2 if sw <= 32 and not local_src else 1
      waves = gk // nbw

      def edge_step(g, _):
        grow = s * (EPAD_SUB // GROUP) + g * gk
        pltpu.sync_copy(src_hbm.at[pl.ds(grow, gk)], sidx)
        pltpu.sync_copy(dst_hbm.at[pl.ds(grow, gk)], didx)
        scat = [[], []]
        for w in range(waves):
          st = w % 2
          for d in scat[st]:
            d.wait()
          scat[st] = []
          gd = []
          for k in range(nbw):
            j = w * nbw + k
            gd.append(pltpu.async_copy(
                src_tab.at[sidx.at[j]], rows.at[st, k], sem))
          for d in gd:
            d.wait()
          for k in range(nbw):
            j = w * nbw + k
            scat[st].append(pltpu.async_copy(
                rows.at[st, k], acc.at[didx.at[j]], ssem, add=True))
        for st in (0, 1):
          for d in scat[st]:
            d.wait()
        return 0
      lax.fori_loop(0, EPAD_SUB // GROUP // gk, edge_step, 0)
      plsc.subcore_barrier()
      pltpu.sync_copy(acc.at[pl.ds(s * w0, w0)],
                      out_hbm.at[sl].at[pl.ds(s * w0, w0)])
      plsc.subcore_barrier()

  return run(x, srcg, dstg)


def _counts_sc(cpg, pcg):
  np_out = _ceil_to(NP_N, GROUP)
  nc_out = _ceil_to(NC_N, GROUP)
  np_pad = _ceil_to(np_out + 1, N_SUB * ZCH)
  nc_pad = _ceil_to(nc_out + 1, N_SUB * ZCH)
  mesh = plsc.VectorSubcoreMesh(core_axis_name="c", subcore_axis_name="s")

  @functools.partial(
      pl.kernel,
      out_type=(jax.ShapeDtypeStruct((np_out, 16), f32),
                jax.ShapeDtypeStruct((nc_out, 16), f32)),
      mesh=mesh,
      scratch_types=[
          pltpu.VMEM_SHARED((np_pad, 16), f32),
          pltpu.VMEM_SHARED((nc_pad, 16), f32),
          pltpu.VMEM((ZCH, 16), f32),
          pltpu.VMEM((GK, GROUP), i32),
          pltpu.VMEM((GROUP, 16), f32),
      ],
      compiler_params=pltpu.CompilerParams(use_tc_tiling_on_sc=False),
  )
  def run(cp_hbm, pc_hbm, outp_hbm, outc_hbm, accp, accc, zbuf, didx, ones):
    c = lax.axis_index("c")
    s = lax.axis_index("s")
    _fill(zbuf, 0.0, ZCH, 16)
    _fill(ones, 1.0, GROUP, 16)

    for core, acc, idx_hbm, out_hbm, n_out, n_pad in (
        (0, accp, cp_hbm, outp_hbm, np_out, np_pad),
        (1, accc, pc_hbm, outc_hbm, nc_out, nc_pad),
    ):
      r0 = n_pad // N_SUB
      w0 = n_out // N_SUB

      @pl.when(c == core)
      def _():
        def zero_step(k, _):
          pltpu.sync_copy(zbuf, acc.at[pl.ds(s * r0 + k * ZCH, ZCH)])
          return 0
        lax.fori_loop(0, r0 // ZCH, zero_step, 0)
        plsc.subcore_barrier()

        def edge_step(g, _):
          grow = s * (EPAD_SUB // GROUP) + g * GK
          pltpu.sync_copy(idx_hbm.at[pl.ds(grow, GK)], didx)
          for j in range(GK):
            pltpu.sync_copy(ones, acc.at[didx.at[j]], add=True)
          return 0
        lax.fori_loop(0, EPAD_SUB // GROUP // GK, edge_step, 0)
        plsc.subcore_barrier()
        pltpu.sync_copy(acc.at[pl.ds(s * w0, w0)], out_hbm.at[pl.ds(s * w0, w0)])

  return run(cpg, pcg)


def kernel(x_country, x_product, edge_src, edge_dst, Wc, bc, Wp, bp,
           W1_cp_l, b1_cp, W1_cp_r, W1_pc_l, b1_pc, W1_pc_r,
           W2_cp_l, b2_cp, W2_cp_r, W2_pc_l, b2_pc, W2_pc_r,
           Wih_c, Whh_c, bih_c, bhh_c, Wih_p, Whh_p, bih_p, bhh_p):
  es = edge_src.astype(i32)
  ed = edge_dst.astype(i32)
  pad = E_PAD - E

  def _g(a, fill):
    return jnp.concatenate([a, jnp.full((pad,), fill, i32)]).reshape(
        N_GROUPS, GROUP)

  cp_src = _g(es, 0)
  cp_dst = _g(ed, NP_N)
  pc_src = _g(ed, 0)
  pc_dst = _g(es, NC_N)

  r1 = lambda b: b.reshape(1, -1)

  cnt_p, cnt_c = _counts_sc(cp_dst, pc_dst)

  hc = _encode(x_country, Wc, r1(bc), NC_N, DC, 2000, slice_major=True)
  hp = _encode(x_product, Wp, r1(bp), NP_N, DP, 2000, slice_major=False)

  seg_p1 = _spmm_sc(hc, cp_src, cp_dst, NC_N, NP_N, 32)
  seg_c1 = _spmm_sc(hp, pc_src, pc_dst, NP_N, NC_N, 128)

  p1 = _combine(seg_p1, cnt_p, hp, W1_cp_l, r1(b1_cp), W1_cp_r,
                NP_N, 2000, relu=True, out_sl=False)
  c1 = _combine(seg_c1, cnt_c, hc, W1_pc_l, r1(b1_pc), W1_pc_r,
                NC_N, 2000, relu=True, out_sl=True)

  seg_p2 = _spmm_sc(c1, cp_src, cp_dst, NC_N, NP_N, 32)
  seg_c2 = _spmm_sc(p1, pc_src, pc_dst, NP_N, NC_N, 128)

  p2 = _combine(seg_p2, cnt_p, p1, W2_cp_l, r1(b2_cp), W2_cp_r,
                NP_N, 2000, relu=False, out_sl=False)
  c2 = _combine(seg_c2, cnt_c, c1, W2_pc_l, r1(b2_pc), W2_pc_r,
                NC_N, 2000, relu=False, out_sl=False)

  zc = _gru(c2, Wih_c, Whh_c, r1(bih_c), r1(bhh_c), NC_N, 2000)
  zp = _gru(p2, Wih_p, Whh_p, r1(bih_p), r1(bhh_p), NP_N, 2000)
  return (zc, zp)


# speedup vs baseline: 2.9171x; 1.1211x over previous
import functools

import jax
import jax.numpy as jnp
from jax import lax
from jax.experimental import pallas as pl
from jax.experimental.pallas import tpu as pltpu
from jax.experimental.pallas import tpu_sc as plsc

T = 4
NC_N = 10000
NP_N = 50000
DC = 64
DP = 32
H = 128
E = 320000

N_CORES = 2
N_SUB = 16
GROUP = 128
GK = 8
EPAD_SUB = 160 * GROUP
E_PAD = N_SUB * EPAD_SUB
N_GROUPS = E_PAD // GROUP
ZCH = 32

f32 = jnp.float32
i32 = jnp.int32


def _ceil_to(x, m):
  return (x + m - 1) // m * m


def _encode_sl_body(x_ref, w_ref, b_ref, o_ref):
  h = jnp.dot(x_ref[0], w_ref[...], preferred_element_type=f32) + b_ref[...]
  for j in range(4):
    o_ref[j] = h[:, 32 * j:32 * (j + 1)]


def _encode_st_body(x_ref, w_ref, b_ref, o_ref):
  o_ref[0] = jnp.dot(x_ref[0], w_ref[...], preferred_element_type=f32) + b_ref[...]


def _encode(x, w, b, n, d, bsz, slice_major):
  body = _encode_sl_body if slice_major else _encode_st_body
  nsl = 16 if slice_major else T
  oblk = (4, bsz, 32) if slice_major else (1, bsz, H)
  return pl.pallas_call(
      body,
      grid=(T, n // bsz),
      in_specs=[
          pl.BlockSpec((1, bsz, d), lambda t, i: (t, i, 0)),
          pl.BlockSpec((d, H), lambda t, i: (0, 0)),
          pl.BlockSpec((1, H), lambda t, i: (0, 0)),
      ],
      out_specs=pl.BlockSpec(oblk, lambda t, i: (t, i, 0)),
      out_shape=jax.ShapeDtypeStruct((nsl, n, oblk[2]), f32),
  )(x, w, b)


def _combine_body(relu, seg_sl, dst_sl, out_sl,
                  seg_ref, cnt_ref, xd_ref, wl_ref, bl_ref, wr_ref, o_ref):
  if seg_sl:
    seg = jnp.concatenate([seg_ref[j] for j in range(4)], axis=1)
  else:
    seg = seg_ref[0]
  if dst_sl:
    xd = jnp.concatenate([xd_ref[j] for j in range(4)], axis=1)
  else:
    xd = xd_ref[0]
  cnt = jnp.maximum(cnt_ref[:, 0:1], 1.0)
  mean = seg / cnt
  h = (jnp.dot(mean, wl_ref[...], preferred_element_type=f32) + bl_ref[...]
       + jnp.dot(xd, wr_ref[...], preferred_element_type=f32))
  if relu:
    h = jnp.maximum(h, 0.0)
  if out_sl:
    for j in range(4):
      o_ref[j] = h[:, 32 * j:32 * (j + 1)]
  else:
    o_ref[0] = h


def _combine(seg, cnt, xd, wl, bl, wr, n, bsz, relu, out_sl):
  seg_sl = seg.shape[0] == 16
  dst_sl = xd.shape[0] == 16
  sblk = (4, bsz, 32) if seg_sl else (1, bsz, H)
  dblk = (4, bsz, 32) if dst_sl else (1, bsz, H)
  oblk = (4, bsz, 32) if out_sl else (1, bsz, H)
  nsl = 16 if out_sl else T
  body = functools.partial(_combine_body, relu, seg_sl, dst_sl, out_sl)
  return pl.pallas_call(
      body,
      grid=(T, n // bsz),
      in_specs=[
          pl.BlockSpec(sblk, lambda t, i: (t, i, 0)),
          pl.BlockSpec((bsz, 16), lambda t, i: (i, 0)),
          pl.BlockSpec(dblk, lambda t, i: (t, i, 0)),
          pl.BlockSpec((H, H), lambda t, i: (0, 0)),
          pl.BlockSpec((1, H), lambda t, i: (0, 0)),
          pl.BlockSpec((H, H), lambda t, i: (0, 0)),
      ],
      out_specs=pl.BlockSpec(oblk, lambda t, i: (t, i, 0)),
      out_shape=jax.ShapeDtypeStruct((nsl, n, oblk[2]), f32),
  )(seg, cnt, xd, wl, bl, wr)


def _gru_body(seq_ref, wih_ref, whh_ref, bih_ref, bhh_ref, o_ref):
  b = seq_ref.shape[1]
  h = jnp.zeros((b, H), f32)
  for t in range(T):
    x = seq_ref[t]
    gi = jnp.dot(x, wih_ref[...], preferred_element_type=f32) + bih_ref[...]
    gh = jnp.dot(h, whh_ref[...], preferred_element_type=f32) + bhh_ref[...]
    r = jax.nn.sigmoid(gi[:, 0:H] + gh[:, 0:H])
    z = jax.nn.sigmoid(gi[:, H:2 * H] + gh[:, H:2 * H])
    nn = jnp.tanh(gi[:, 2 * H:] + r * gh[:, 2 * H:])
    h = (1.0 - z) * nn + z * h
  o_ref[...] = h


def _gru(seq, wih, whh, bih, bhh, n, bsz):
  return pl.pallas_call(
      _gru_body,
      grid=(n // bsz,),
      in_specs=[
          pl.BlockSpec((T, bsz, H), lambda i: (0, i, 0)),
          pl.BlockSpec((H, 3 * H), lambda i: (0, 0)),
          pl.BlockSpec((H, 3 * H), lambda i: (0, 0)),
          pl.BlockSpec((1, 3 * H), lambda i: (0, 0)),
          pl.BlockSpec((1, 3 * H), lambda i: (0, 0)),
      ],
      out_specs=pl.BlockSpec((bsz, H), lambda i: (i, 0)),
      out_shape=jax.ShapeDtypeStruct((n, H), f32),
  )(seq, wih, whh, bih, bhh)


def _fill(buf, val, nrows, sw):
  v = jnp.full((16,), val, f32)
  def row(r, _):
    for k in range(sw // 16):
      buf[r, pl.ds(16 * k, 16)] = v
    return 0
  lax.fori_loop(0, nrows, row, 0)


def _spmm_sc(x, srcg, dstg, n_src, n_dst, sw):
  nsl = 512 // sw
  passes = nsl // N_CORES
  local_src = n_src * sw <= 400_000
  gk = 8 if local_src else 16
  n_out = _ceil_to(n_dst, GROUP)
  n_dst_pad = _ceil_to(max(n_out, n_dst + 1), N_SUB * 8)
  r0 = n_dst_pad // N_SUB
  zc = r0 // GROUP
  zr = r0 % GROUP
  w0 = n_out // N_SUB
  t0 = n_src // N_SUB
  mesh = plsc.VectorSubcoreMesh(core_axis_name="c", subcore_axis_name="s")

  scratch = [
      pltpu.VMEM_SHARED((n_dst_pad, sw), f32),
      pltpu.VMEM((gk, GROUP), i32),
      pltpu.VMEM((gk, GROUP), i32),
      pltpu.VMEM((2, GROUP, sw), f32),
      pltpu.SemaphoreType.DMA,
      pltpu.SemaphoreType.DMA,
      pltpu.SemaphoreType.DMA,
  ]
  if local_src:
    scratch.append(pltpu.VMEM_SHARED((n_src, sw), f32))

  @functools.partial(
      pl.kernel,
      out_type=jax.ShapeDtypeStruct((nsl, n_out, sw), f32),
      mesh=mesh,
      scratch_types=scratch,
      compiler_params=pltpu.CompilerParams(use_tc_tiling_on_sc=False),
  )
  def run(x_hbm, src_hbm, dst_hbm, out_hbm, acc, sidx, didx, rows, sem,
          ssem, tsem, *tbl):
    c = lax.axis_index("c")
    s = lax.axis_index("s")
    for p in range(passes):
      sl = c * passes + p
      td = None
      if local_src:
        td = pltpu.async_copy(x_hbm.at[sl].at[pl.ds(s * t0, t0)],
                              tbl[0].at[pl.ds(s * t0, t0)], tsem)
      _fill(rows.at[0], 0.0, GROUP, sw)
      def zero_step(k, _):
        pltpu.sync_copy(rows.at[0], acc.at[pl.ds(s * r0 + k * GROUP, GROUP)])
        return 0
      lax.fori_loop(0, zc, zero_step, 0)
      if zr:
        pltpu.sync_copy(rows.at[0, pl.ds(0, zr)],
                        acc.at[pl.ds(s * r0 + zc * GROUP, zr)])
      if td is not None:
        td.wait()
      plsc.subcore_barrier()
      src_tab = tbl[0] if local_src else x_hbm.at[sl]

      def edge_step(g, _):
        grow = s * (EPAD_SUB // GROUP) + g * gk
        pltpu.sync_copy(src_hbm.at[pl.ds(grow, gk)], sidx)
        pltpu.sync_copy(dst_hbm.at[pl.ds(grow, gk)], didx)
        gd = [None, None]
        scat = [None, None]
        for w in range(gk):
          st = w % 2
          if scat[st] is not None:
            scat[st].wait()
          gd[st] = pltpu.async_copy(src_tab.at[sidx.at[w]], rows.at[st], sem)
          if w > 0:
            pv = (w - 1) % 2
            gd[pv].wait()
            scat[pv] = pltpu.async_copy(
                rows.at[pv], acc.at[didx.at[w - 1]], ssem, add=True)
        last = (gk - 1) % 2
        gd[last].wait()
        scat[last] = pltpu.async_copy(
            rows.at[last], acc.at[didx.at[gk - 1]], ssem, add=True)
        scat[0].wait()
        scat[1].wait()
        return 0
      lax.fori_loop(0, EPAD_SUB // GROUP // gk, edge_step, 0)
      plsc.subcore_barrier()
      pltpu.sync_copy(acc.at[pl.ds(s * w0, w0)],
                      out_hbm.at[sl].at[pl.ds(s * w0, w0)])
      plsc.subcore_barrier()

  return run(x, srcg, dstg)


def _counts_sc(cpg, pcg):
  np_out = _ceil_to(NP_N, GROUP)
  nc_out = _ceil_to(NC_N, GROUP)
  np_pad = _ceil_to(np_out + 1, N_SUB * ZCH)
  nc_pad = _ceil_to(nc_out + 1, N_SUB * ZCH)
  mesh = plsc.VectorSubcoreMesh(core_axis_name="c", subcore_axis_name="s")

  @functools.partial(
      pl.kernel,
      out_type=(jax.ShapeDtypeStruct((np_out, 16), f32),
                jax.ShapeDtypeStruct((nc_out, 16), f32)),
      mesh=mesh,
      scratch_types=[
          pltpu.VMEM_SHARED((np_pad, 16), f32),
          pltpu.VMEM_SHARED((nc_pad, 16), f32),
          pltpu.VMEM((ZCH, 16), f32),
          pltpu.VMEM((GK, GROUP), i32),
          pltpu.VMEM((GROUP, 16), f32),
      ],
      compiler_params=pltpu.CompilerParams(use_tc_tiling_on_sc=False),
  )
  def run(cp_hbm, pc_hbm, outp_hbm, outc_hbm, accp, accc, zbuf, didx, ones):
    c = lax.axis_index("c")
    s = lax.axis_index("s")
    _fill(zbuf, 0.0, ZCH, 16)
    _fill(ones, 1.0, GROUP, 16)

    for core, acc, idx_hbm, out_hbm, n_out, n_pad in (
        (0, accp, cp_hbm, outp_hbm, np_out, np_pad),
        (1, accc, pc_hbm, outc_hbm, nc_out, nc_pad),
    ):
      r0 = n_pad // N_SUB
      w0 = n_out // N_SUB

      @pl.when(c == core)
      def _():
        def zero_step(k, _):
          pltpu.sync_copy(zbuf, acc.at[pl.ds(s * r0 + k * ZCH, ZCH)])
          return 0
        lax.fori_loop(0, r0 // ZCH, zero_step, 0)
        plsc.subcore_barrier()

        def edge_step(g, _):
          grow = s * (EPAD_SUB // GROUP) + g * GK
          pltpu.sync_copy(idx_hbm.at[pl.ds(grow, GK)], didx)
          for j in range(GK):
            pltpu.sync_copy(ones, acc.at[didx.at[j]], add=True)
          return 0
        lax.fori_loop(0, EPAD_SUB // GROUP // GK, edge_step, 0)
        plsc.subcore_barrier()
        pltpu.sync_copy(acc.at[pl.ds(s * w0, w0)], out_hbm.at[pl.ds(s * w0, w0)])

  return run(cpg, pcg)


def kernel(x_country, x_product, edge_src, edge_dst, Wc, bc, Wp, bp,
           W1_cp_l, b1_cp, W1_cp_r, W1_pc_l, b1_pc, W1_pc_r,
           W2_cp_l, b2_cp, W2_cp_r, W2_pc_l, b2_pc, W2_pc_r,
           Wih_c, Whh_c, bih_c, bhh_c, Wih_p, Whh_p, bih_p, bhh_p):
  es = edge_src.astype(i32)
  ed = edge_dst.astype(i32)
  pad = E_PAD - E

  def _g(a, fill):
    return jnp.concatenate([a, jnp.full((pad,), fill, i32)]).reshape(
        N_GROUPS, GROUP)

  cp_src = _g(es, 0)
  cp_dst = _g(ed, NP_N)
  pc_src = _g(ed, 0)
  pc_dst = _g(es, NC_N)

  r1 = lambda b: b.reshape(1, -1)

  cnt_p, cnt_c = _counts_sc(cp_dst, pc_dst)

  hc = _encode(x_country, Wc, r1(bc), NC_N, DC, 2000, slice_major=True)
  hp = _encode(x_product, Wp, r1(bp), NP_N, DP, 2000, slice_major=False)

  seg_p1 = _spmm_sc(hc, cp_src, cp_dst, NC_N, NP_N, 32)
  seg_c1 = _spmm_sc(hp, pc_src, pc_dst, NP_N, NC_N, 128)

  p1 = _combine(seg_p1, cnt_p, hp, W1_cp_l, r1(b1_cp), W1_cp_r,
                NP_N, 2000, relu=True, out_sl=False)
  c1 = _combine(seg_c1, cnt_c, hc, W1_pc_l, r1(b1_pc), W1_pc_r,
                NC_N, 2000, relu=True, out_sl=True)

  seg_p2 = _spmm_sc(c1, cp_src, cp_dst, NC_N, NP_N, 32)
  seg_c2 = _spmm_sc(p1, pc_src, pc_dst, NP_N, NC_N, 128)

  p2 = _combine(seg_p2, cnt_p, p1, W2_cp_l, r1(b2_cp), W2_cp_r,
                NP_N, 2000, relu=False, out_sl=False)
  c2 = _combine(seg_c2, cnt_c, c1, W2_pc_l, r1(b2_pc), W2_pc_r,
                NC_N, 2000, relu=False, out_sl=False)

  zc = _gru(c2, Wih_c, Whh_c, r1(bih_c), r1(bhh_c), NC_N, 2000)
  zp = _gru(p2, Wih_p, Whh_p, r1(bih_p), r1(bhh_p), NP_N, 2000)
  return (zc, zp)


# speedup vs baseline: 3.0331x; 1.0398x over previous
import functools

import jax
import jax.numpy as jnp
from jax import lax
from jax.experimental import pallas as pl
from jax.experimental.pallas import tpu as pltpu
from jax.experimental.pallas import tpu_sc as plsc

T = 4
NC_N = 10000
NP_N = 50000
DC = 64
DP = 32
H = 128
E = 320000

N_CORES = 2
N_SUB = 16
GROUP = 128
GK = 8
EPAD_SUB = 160 * GROUP
E_PAD = N_SUB * EPAD_SUB
N_GROUPS = E_PAD // GROUP
ZCH = 32

f32 = jnp.float32
i32 = jnp.int32


def _ceil_to(x, m):
  return (x + m - 1) // m * m


def _encode_sl_body(x_ref, w_ref, b_ref, o_ref):
  h = jnp.dot(x_ref[0], w_ref[...], preferred_element_type=f32) + b_ref[...]
  for j in range(4):
    o_ref[j] = h[:, 32 * j:32 * (j + 1)]


def _encode_st_body(x_ref, w_ref, b_ref, o_ref):
  o_ref[0] = jnp.dot(x_ref[0], w_ref[...], preferred_element_type=f32) + b_ref[...]


def _encode(x, w, b, n, d, bsz, slice_major):
  body = _encode_sl_body if slice_major else _encode_st_body
  nsl = 16 if slice_major else T
  oblk = (4, bsz, 32) if slice_major else (1, bsz, H)
  return pl.pallas_call(
      body,
      grid=(T, n // bsz),
      in_specs=[
          pl.BlockSpec((1, bsz, d), lambda t, i: (t, i, 0)),
          pl.BlockSpec((d, H), lambda t, i: (0, 0)),
          pl.BlockSpec((1, H), lambda t, i: (0, 0)),
      ],
      out_specs=pl.BlockSpec(oblk, lambda t, i: (t, i, 0)),
      out_shape=jax.ShapeDtypeStruct((nsl, n, oblk[2]), f32),
  )(x, w, b)


def _combine_body(relu, seg_sl, dst_sl, out_sl,
                  seg_ref, cnt_ref, xd_ref, wl_ref, bl_ref, wr_ref, o_ref):
  if seg_sl:
    seg = jnp.concatenate([seg_ref[j] for j in range(4)], axis=1)
  else:
    seg = seg_ref[0]
  if dst_sl:
    xd = jnp.concatenate([xd_ref[j] for j in range(4)], axis=1)
  else:
    xd = xd_ref[0]
  cnt = jnp.maximum(cnt_ref[:, 0:1], 1.0)
  mean = seg / cnt
  h = (jnp.dot(mean, wl_ref[...], preferred_element_type=f32) + bl_ref[...]
       + jnp.dot(xd, wr_ref[...], preferred_element_type=f32))
  if relu:
    h = jnp.maximum(h, 0.0)
  if out_sl:
    for j in range(4):
      o_ref[j] = h[:, 32 * j:32 * (j + 1)]
  else:
    o_ref[0] = h


def _combine(seg, cnt, xd, wl, bl, wr, n, bsz, relu, out_sl):
  seg_sl = seg.shape[0] == 16
  dst_sl = xd.shape[0] == 16
  sblk = (4, bsz, 32) if seg_sl else (1, bsz, H)
  dblk = (4, bsz, 32) if dst_sl else (1, bsz, H)
  oblk = (4, bsz, 32) if out_sl else (1, bsz, H)
  nsl = 16 if out_sl else T
  body = functools.partial(_combine_body, relu, seg_sl, dst_sl, out_sl)
  return pl.pallas_call(
      body,
      grid=(T, n // bsz),
      in_specs=[
          pl.BlockSpec(sblk, lambda t, i: (t, i, 0)),
          pl.BlockSpec((bsz, 16), lambda t, i: (i, 0)),
          pl.BlockSpec(dblk, lambda t, i: (t, i, 0)),
          pl.BlockSpec((H, H), lambda t, i: (0, 0)),
          pl.BlockSpec((1, H), lambda t, i: (0, 0)),
          pl.BlockSpec((H, H), lambda t, i: (0, 0)),
      ],
      out_specs=pl.BlockSpec(oblk, lambda t, i: (t, i, 0)),
      out_shape=jax.ShapeDtypeStruct((nsl, n, oblk[2]), f32),
  )(seg, cnt, xd, wl, bl, wr)


def _gru_body(seq_ref, wih_ref, whh_ref, bih_ref, bhh_ref, o_ref):
  b = seq_ref.shape[1]
  h = jnp.zeros((b, H), f32)
  for t in range(T):
    x = seq_ref[t]
    gi = jnp.dot(x, wih_ref[...], preferred_element_type=f32) + bih_ref[...]
    gh = jnp.dot(h, whh_ref[...], preferred_element_type=f32) + bhh_ref[...]
    r = jax.nn.sigmoid(gi[:, 0:H] + gh[:, 0:H])
    z = jax.nn.sigmoid(gi[:, H:2 * H] + gh[:, H:2 * H])
    nn = jnp.tanh(gi[:, 2 * H:] + r * gh[:, 2 * H:])
    h = (1.0 - z) * nn + z * h
  o_ref[...] = h


def _gru(seq, wih, whh, bih, bhh, n, bsz):
  return pl.pallas_call(
      _gru_body,
      grid=(n // bsz,),
      in_specs=[
          pl.BlockSpec((T, bsz, H), lambda i: (0, i, 0)),
          pl.BlockSpec((H, 3 * H), lambda i: (0, 0)),
          pl.BlockSpec((H, 3 * H), lambda i: (0, 0)),
          pl.BlockSpec((1, 3 * H), lambda i: (0, 0)),
          pl.BlockSpec((1, 3 * H), lambda i: (0, 0)),
      ],
      out_specs=pl.BlockSpec((bsz, H), lambda i: (i, 0)),
      out_shape=jax.ShapeDtypeStruct((n, H), f32),
  )(seq, wih, whh, bih, bhh)


def _fill(buf, val, nrows, sw):
  v = jnp.full((16,), val, f32)
  def row(r, _):
    for k in range(sw // 16):
      buf[r, pl.ds(16 * k, 16)] = v
    return 0
  lax.fori_loop(0, nrows, row, 0)


def _spmm_sc(x, eg, n_src, n_dst, sw):
  nsl = 512 // sw
  passes = nsl // N_CORES
  local_src = n_src * sw <= 400_000
  gk = 5 if local_src else 20
  steps = EPAD_SUB // GROUP // gk
  n_out = _ceil_to(n_dst, GROUP)
  n_dst_pad = _ceil_to(max(n_out, n_dst + 1), N_SUB * 8)
  r0 = n_dst_pad // N_SUB
  zc = r0 // GROUP
  zr = r0 % GROUP
  w0 = n_out // N_SUB
  t0 = n_src // N_SUB
  mesh = plsc.VectorSubcoreMesh(core_axis_name="c", subcore_axis_name="s")

  scratch = [
      pltpu.VMEM_SHARED((n_dst_pad, sw), f32),
      pltpu.VMEM((2, gk, 2, GROUP), i32),
      pltpu.VMEM((2, GROUP, sw), f32),
      pltpu.SemaphoreType.DMA,
      pltpu.SemaphoreType.DMA,
      pltpu.SemaphoreType.DMA,
      pltpu.SemaphoreType.DMA,
  ]
  if local_src:
    scratch.append(pltpu.VMEM_SHARED((n_src, sw), f32))

  @functools.partial(
      pl.kernel,
      out_type=jax.ShapeDtypeStruct((nsl, n_out, sw), f32),
      mesh=mesh,
      scratch_types=scratch,
      compiler_params=pltpu.CompilerParams(use_tc_tiling_on_sc=False),
  )
  def run(x_hbm, eg_hbm, out_hbm, acc, idx, rows, sem, ssem, isem, tsem,
          *tbl):
    c = lax.axis_index("c")
    s = lax.axis_index("s")
    for p in range(passes):
      sl = c * passes + p
      base = s * (EPAD_SUB // GROUP)
      pltpu.async_copy(eg_hbm.at[pl.ds(base, gk)], idx.at[0], isem)
      td = None
      if local_src:
        td = pltpu.async_copy(x_hbm.at[sl].at[pl.ds(s * t0, t0)],
                              tbl[0].at[pl.ds(s * t0, t0)], tsem)
      _fill(rows.at[0], 0.0, GROUP, sw)
      def zero_step(k, _):
        pltpu.sync_copy(rows.at[0], acc.at[pl.ds(s * r0 + k * GROUP, GROUP)])
        return 0
      lax.fori_loop(0, zc, zero_step, 0)
      if zr:
        pltpu.sync_copy(rows.at[0, pl.ds(0, zr)],
                        acc.at[pl.ds(s * r0 + zc * GROUP, zr)])
      if td is not None:
        td.wait()
      plsc.subcore_barrier()
      src_tab = tbl[0] if local_src else x_hbm.at[sl]

      def edge_step(g, _):
        b = lax.rem(g, 2)
        grow = base + g * gk
        pltpu.make_async_copy(eg_hbm.at[pl.ds(grow, gk)], idx.at[b],
                              isem).wait()
        @pl.when(g < steps - 1)
        def _():
          pltpu.async_copy(eg_hbm.at[pl.ds(grow + gk, gk)], idx.at[1 - b],
                           isem)
        gd = [None, None]
        scat = [None, None]
        for w in range(gk):
          st = w % 2
          if scat[st] is not None:
            scat[st].wait()
          gd[st] = pltpu.async_copy(src_tab.at[idx.at[b, w, 0]], rows.at[st],
                                    sem)
          if w > 0:
            pv = (w - 1) % 2
            gd[pv].wait()
            scat[pv] = pltpu.async_copy(
                rows.at[pv], acc.at[idx.at[b, w - 1, 1]], ssem, add=True)
        last = (gk - 1) % 2
        gd[last].wait()
        scat[last] = pltpu.async_copy(
            rows.at[last], acc.at[idx.at[b, gk - 1, 1]], ssem, add=True)
        scat[0].wait()
        scat[1].wait()
        return 0
      lax.fori_loop(0, steps, edge_step, 0)
      plsc.subcore_barrier()
      pltpu.sync_copy(acc.at[pl.ds(s * w0, w0)],
                      out_hbm.at[sl].at[pl.ds(s * w0, w0)])
      plsc.subcore_barrier()

  return run(x, eg)


def _counts_sc(cpg, pcg):
  np_out = _ceil_to(NP_N, GROUP)
  nc_out = _ceil_to(NC_N, GROUP)
  np_pad = _ceil_to(np_out + 1, N_SUB * ZCH)
  nc_pad = _ceil_to(nc_out + 1, N_SUB * ZCH)
  mesh = plsc.VectorSubcoreMesh(core_axis_name="c", subcore_axis_name="s")

  @functools.partial(
      pl.kernel,
      out_type=(jax.ShapeDtypeStruct((np_out, 16), f32),
                jax.ShapeDtypeStruct((nc_out, 16), f32)),
      mesh=mesh,
      scratch_types=[
          pltpu.VMEM_SHARED((np_pad, 16), f32),
          pltpu.VMEM_SHARED((nc_pad, 16), f32),
          pltpu.VMEM((ZCH, 16), f32),
          pltpu.VMEM((GK, GROUP), i32),
          pltpu.VMEM((GROUP, 16), f32),
      ],
      compiler_params=pltpu.CompilerParams(use_tc_tiling_on_sc=False),
  )
  def run(cp_hbm, pc_hbm, outp_hbm, outc_hbm, accp, accc, zbuf, didx, ones):
    c = lax.axis_index("c")
    s = lax.axis_index("s")
    _fill(zbuf, 0.0, ZCH, 16)
    _fill(ones, 1.0, GROUP, 16)

    for core, acc, idx_hbm, out_hbm, n_out, n_pad in (
        (0, accp, cp_hbm, outp_hbm, np_out, np_pad),
        (1, accc, pc_hbm, outc_hbm, nc_out, nc_pad),
    ):
      r0 = n_pad // N_SUB
      w0 = n_out // N_SUB

      @pl.when(c == core)
      def _():
        def zero_step(k, _):
          pltpu.sync_copy(zbuf, acc.at[pl.ds(s * r0 + k * ZCH, ZCH)])
          return 0
        lax.fori_loop(0, r0 // ZCH, zero_step, 0)
        plsc.subcore_barrier()

        def edge_step(g, _):
          grow = s * (EPAD_SUB // GROUP) + g * GK
          pltpu.sync_copy(idx_hbm.at[pl.ds(grow, GK)], didx)
          for j in range(GK):
            pltpu.sync_copy(ones, acc.at[didx.at[j]], add=True)
          return 0
        lax.fori_loop(0, EPAD_SUB // GROUP // GK, edge_step, 0)
        plsc.subcore_barrier()
        pltpu.sync_copy(acc.at[pl.ds(s * w0, w0)], out_hbm.at[pl.ds(s * w0, w0)])

  return run(cpg, pcg)


def kernel(x_country, x_product, edge_src, edge_dst, Wc, bc, Wp, bp,
           W1_cp_l, b1_cp, W1_cp_r, W1_pc_l, b1_pc, W1_pc_r,
           W2_cp_l, b2_cp, W2_cp_r, W2_pc_l, b2_pc, W2_pc_r,
           Wih_c, Whh_c, bih_c, bhh_c, Wih_p, Whh_p, bih_p, bhh_p):
  es = edge_src.astype(i32)
  ed = edge_dst.astype(i32)
  pad = E_PAD - E

  def _g(a, fill):
    return jnp.concatenate([a, jnp.full((pad,), fill, i32)]).reshape(
        N_GROUPS, GROUP)

  cp_src = _g(es, 0)
  cp_dst = _g(ed, NP_N)
  pc_src = _g(ed, 0)
  pc_dst = _g(es, NC_N)
  eg_cp = jnp.stack([cp_src, cp_dst], axis=1)
  eg_pc = jnp.stack([pc_src, pc_dst], axis=1)

  r1 = lambda b: b.reshape(1, -1)

  cnt_p, cnt_c = _counts_sc(cp_dst, pc_dst)

  hc = _encode(x_country, Wc, r1(bc), NC_N, DC, 2000, slice_major=True)
  hp = _encode(x_product, Wp, r1(bp), NP_N, DP, 2000, slice_major=False)

  seg_p1 = _spmm_sc(hc, eg_cp, NC_N, NP_N, 32)
  seg_c1 = _spmm_sc(hp, eg_pc, NP_N, NC_N, 128)

  p1 = _combine(seg_p1, cnt_p, hp, W1_cp_l, r1(b1_cp), W1_cp_r,
                NP_N, 2000, relu=True, out_sl=False)
  c1 = _combine(seg_c1, cnt_c, hc, W1_pc_l, r1(b1_pc), W1_pc_r,
                NC_N, 2000, relu=True, out_sl=True)

  seg_p2 = _spmm_sc(c1, eg_cp, NC_N, NP_N, 32)
  seg_c2 = _spmm_sc(p1, eg_pc, NP_N, NC_N, 128)

  p2 = _combine(seg_p2, cnt_p, p1, W2_cp_l, r1(b2_cp), W2_cp_r,
                NP_N, 2000, relu=False, out_sl=False)
  c2 = _combine(seg_c2, cnt_c, c1, W2_pc_l, r1(b2_pc), W2_pc_r,
                NC_N, 2000, relu=False, out_sl=False)

  zc = _gru(c2, Wih_c, Whh_c, r1(bih_c), r1(bhh_c), NC_N, 2000)
  zp = _gru(p2, Wih_p, Whh_p, r1(bih_p), r1(bhh_p), NP_N, 2000)
  return (zc, zp)


# speedup vs baseline: 3.0371x; 1.0013x over previous
import functools

import jax
import jax.numpy as jnp
from jax import lax
from jax.experimental import pallas as pl
from jax.experimental.pallas import tpu as pltpu
from jax.experimental.pallas import tpu_sc as plsc

T = 4
NC_N = 10000
NP_N = 50000
DC = 64
DP = 32
H = 128
E = 320000

N_CORES = 2
N_SUB = 16
GROUP = 128
GK = 8
EPAD_SUB = 160 * GROUP
E_PAD = N_SUB * EPAD_SUB
N_GROUPS = E_PAD // GROUP
ZCH = 32

f32 = jnp.float32
i32 = jnp.int32


def _ceil_to(x, m):
  return (x + m - 1) // m * m


def _encode_sl_body(x_ref, w_ref, b_ref, o_ref):
  h = jnp.dot(x_ref[0], w_ref[...], preferred_element_type=f32) + b_ref[...]
  for j in range(4):
    o_ref[j] = h[:, 32 * j:32 * (j + 1)]


def _encode_st_body(x_ref, w_ref, b_ref, o_ref):
  o_ref[0] = jnp.dot(x_ref[0], w_ref[...], preferred_element_type=f32) + b_ref[...]


def _encode(x, w, b, n, d, bsz, slice_major):
  body = _encode_sl_body if slice_major else _encode_st_body
  nsl = 16 if slice_major else T
  oblk = (4, bsz, 32) if slice_major else (1, bsz, H)
  return pl.pallas_call(
      body,
      grid=(T, n // bsz),
      in_specs=[
          pl.BlockSpec((1, bsz, d), lambda t, i: (t, i, 0)),
          pl.BlockSpec((d, H), lambda t, i: (0, 0)),
          pl.BlockSpec((1, H), lambda t, i: (0, 0)),
      ],
      out_specs=pl.BlockSpec(oblk, lambda t, i: (t, i, 0)),
      out_shape=jax.ShapeDtypeStruct((nsl, n, oblk[2]), f32),
  )(x, w, b)


def _combine_body(relu, seg_sl, dst_sl, out_sl,
                  seg_ref, cnt_ref, xd_ref, wl_ref, bl_ref, wr_ref, o_ref):
  if seg_sl:
    seg = jnp.concatenate([seg_ref[j] for j in range(4)], axis=1)
  else:
    seg = seg_ref[0]
  if dst_sl:
    xd = jnp.concatenate([xd_ref[j] for j in range(4)], axis=1)
  else:
    xd = xd_ref[0]
  cnt = jnp.maximum(cnt_ref[:, 0:1], 1.0)
  mean = seg / cnt
  h = (jnp.dot(mean, wl_ref[...], preferred_element_type=f32) + bl_ref[...]
       + jnp.dot(xd, wr_ref[...], preferred_element_type=f32))
  if relu:
    h = jnp.maximum(h, 0.0)
  if out_sl:
    for j in range(4):
      o_ref[j] = h[:, 32 * j:32 * (j + 1)]
  else:
    o_ref[0] = h


def _combine(seg, cnt, xd, wl, bl, wr, n, bsz, relu, out_sl):
  seg_sl = seg.shape[0] == 16
  dst_sl = xd.shape[0] == 16
  sblk = (4, bsz, 32) if seg_sl else (1, bsz, H)
  dblk = (4, bsz, 32) if dst_sl else (1, bsz, H)
  oblk = (4, bsz, 32) if out_sl else (1, bsz, H)
  nsl = 16 if out_sl else T
  body = functools.partial(_combine_body, relu, seg_sl, dst_sl, out_sl)
  return pl.pallas_call(
      body,
      grid=(T, n // bsz),
      in_specs=[
          pl.BlockSpec(sblk, lambda t, i: (t, i, 0)),
          pl.BlockSpec((bsz, 16), lambda t, i: (i, 0)),
          pl.BlockSpec(dblk, lambda t, i: (t, i, 0)),
          pl.BlockSpec((H, H), lambda t, i: (0, 0)),
          pl.BlockSpec((1, H), lambda t, i: (0, 0)),
          pl.BlockSpec((H, H), lambda t, i: (0, 0)),
      ],
      out_specs=pl.BlockSpec(oblk, lambda t, i: (t, i, 0)),
      out_shape=jax.ShapeDtypeStruct((nsl, n, oblk[2]), f32),
  )(seg, cnt, xd, wl, bl, wr)


def _gru_body(seq_ref, wih_ref, whh_ref, bih_ref, bhh_ref, o_ref):
  b = seq_ref.shape[1]
  h = jnp.zeros((b, H), f32)
  for t in range(T):
    x = seq_ref[t]
    gi = jnp.dot(x, wih_ref[...], preferred_element_type=f32) + bih_ref[...]
    gh = jnp.dot(h, whh_ref[...], preferred_element_type=f32) + bhh_ref[...]
    r = jax.nn.sigmoid(gi[:, 0:H] + gh[:, 0:H])
    z = jax.nn.sigmoid(gi[:, H:2 * H] + gh[:, H:2 * H])
    nn = jnp.tanh(gi[:, 2 * H:] + r * gh[:, 2 * H:])
    h = (1.0 - z) * nn + z * h
  o_ref[...] = h


def _gru(seq, wih, whh, bih, bhh, n, bsz):
  return pl.pallas_call(
      _gru_body,
      grid=(n // bsz,),
      in_specs=[
          pl.BlockSpec((T, bsz, H), lambda i: (0, i, 0)),
          pl.BlockSpec((H, 3 * H), lambda i: (0, 0)),
          pl.BlockSpec((H, 3 * H), lambda i: (0, 0)),
          pl.BlockSpec((1, 3 * H), lambda i: (0, 0)),
          pl.BlockSpec((1, 3 * H), lambda i: (0, 0)),
      ],
      out_specs=pl.BlockSpec((bsz, H), lambda i: (i, 0)),
      out_shape=jax.ShapeDtypeStruct((n, H), f32),
  )(seq, wih, whh, bih, bhh)


def _fill(buf, val, nrows, sw):
  v = jnp.full((16,), val, f32)
  def row(r, _):
    for k in range(sw // 16):
      buf[r, pl.ds(16 * k, 16)] = v
    return 0
  lax.fori_loop(0, nrows, row, 0)


def _spmm_sc(x, eg, n_src, n_dst, sw):
  nsl = 512 // sw
  passes = nsl // N_CORES
  local_src = n_src * sw <= 400_000
  gk = 5 if local_src else 32
  nb = 2
  hsplit = not local_src
  steps = EPAD_SUB // GROUP // gk
  n_out = _ceil_to(n_dst, GROUP)
  n_dst_pad = _ceil_to(max(n_out, n_dst + 1), N_SUB * 8)
  r0 = n_dst_pad // N_SUB
  zc = r0 // GROUP
  zr = r0 % GROUP
  w0 = n_out // N_SUB
  t0 = n_src // N_SUB
  mesh = plsc.VectorSubcoreMesh(core_axis_name="c", subcore_axis_name="s")

  scratch = [
      pltpu.VMEM_SHARED((n_dst_pad, sw), f32),
      pltpu.VMEM((2, gk, 2, GROUP), i32),
      pltpu.VMEM((nb, GROUP, sw), f32),
      pltpu.SemaphoreType.DMA,
      pltpu.SemaphoreType.DMA,
      pltpu.SemaphoreType.DMA,
      pltpu.SemaphoreType.DMA,
  ]
  if local_src:
    scratch.append(pltpu.VMEM_SHARED((n_src, sw), f32))

  @functools.partial(
      pl.kernel,
      out_type=jax.ShapeDtypeStruct((nsl, n_out, sw), f32),
      mesh=mesh,
      scratch_types=scratch,
      compiler_params=pltpu.CompilerParams(use_tc_tiling_on_sc=False),
  )
  def run(x_hbm, eg_hbm, out_hbm, acc, idx, rows, sem, ssem, isem, tsem,
          *tbl):
    c = lax.axis_index("c")
    s = lax.axis_index("s")
    for p in range(passes):
      sl = c * passes + p
      base = s * (EPAD_SUB // GROUP)
      pltpu.async_copy(eg_hbm.at[pl.ds(base, gk)], idx.at[0], isem)
      td = None
      if local_src:
        td = pltpu.async_copy(x_hbm.at[sl].at[pl.ds(s * t0, t0)],
                              tbl[0].at[pl.ds(s * t0, t0)], tsem)
      _fill(rows.at[0], 0.0, GROUP, sw)
      def zero_step(k, _):
        pltpu.sync_copy(rows.at[0], acc.at[pl.ds(s * r0 + k * GROUP, GROUP)])
        return 0
      lax.fori_loop(0, zc, zero_step, 0)
      if zr:
        pltpu.sync_copy(rows.at[0, pl.ds(0, zr)],
                        acc.at[pl.ds(s * r0 + zc * GROUP, zr)])
      if td is not None:
        td.wait()
      plsc.subcore_barrier()
      src_tab = tbl[0] if local_src else x_hbm.at[sl]

      def edge_step(g, _):
        b = lax.rem(g, 2)
        grow = base + g * gk
        pltpu.make_async_copy(eg_hbm.at[pl.ds(grow, gk)], idx.at[b],
                              isem).wait()
        @pl.when(g < steps - 1)
        def _():
          pltpu.async_copy(eg_hbm.at[pl.ds(grow + gk, gk)], idx.at[1 - b],
                           isem)
        dep = nb - 1
        gd = [None] * nb
        scat = [None] * nb
        hg = GROUP // 2
        for w in range(gk + dep):
          if w < gk:
            st = w % nb
            if scat[st] is not None:
              scat[st].wait()
              scat[st] = None
            if hsplit:
              gd[st] = [
                  pltpu.async_copy(src_tab.at[idx.at[b, w, 0, pl.ds(0, hg)]],
                                   rows.at[st, pl.ds(0, hg)], sem),
                  pltpu.async_copy(src_tab.at[idx.at[b, w, 0, pl.ds(hg, hg)]],
                                   rows.at[st, pl.ds(hg, hg)], sem),
              ]
            else:
              gd[st] = [pltpu.async_copy(src_tab.at[idx.at[b, w, 0]],
                                         rows.at[st], sem)]
          if w >= dep:
            pv = (w - dep) % nb
            for g_ in gd[pv]:
              g_.wait()
            scat[pv] = pltpu.async_copy(
                rows.at[pv], acc.at[idx.at[b, w - dep, 1]], ssem, add=True)
        for st in range(nb):
          if scat[st] is not None:
            scat[st].wait()
        return 0
      lax.fori_loop(0, steps, edge_step, 0)
      plsc.subcore_barrier()
      pltpu.sync_copy(acc.at[pl.ds(s * w0, w0)],
                      out_hbm.at[sl].at[pl.ds(s * w0, w0)])
      plsc.subcore_barrier()

  return run(x, eg)


def _counts_sc(cpg, pcg):
  np_out = _ceil_to(NP_N, GROUP)
  nc_out = _ceil_to(NC_N, GROUP)
  np_pad = _ceil_to(np_out + 1, N_SUB * ZCH)
  nc_pad = _ceil_to(nc_out + 1, N_SUB * ZCH)
  mesh = plsc.VectorSubcoreMesh(core_axis_name="c", subcore_axis_name="s")

  @functools.partial(
      pl.kernel,
      out_type=(jax.ShapeDtypeStruct((np_out, 16), f32),
                jax.ShapeDtypeStruct((nc_out, 16), f32)),
      mesh=mesh,
      scratch_types=[
          pltpu.VMEM_SHARED((np_pad, 16), f32),
          pltpu.VMEM_SHARED((nc_pad, 16), f32),
          pltpu.VMEM((ZCH, 16), f32),
          pltpu.VMEM((GK, GROUP), i32),
          pltpu.VMEM((GROUP, 16), f32),
      ],
      compiler_params=pltpu.CompilerParams(use_tc_tiling_on_sc=False),
  )
  def run(cp_hbm, pc_hbm, outp_hbm, outc_hbm, accp, accc, zbuf, didx, ones):
    c = lax.axis_index("c")
    s = lax.axis_index("s")
    _fill(zbuf, 0.0, ZCH, 16)
    _fill(ones, 1.0, GROUP, 16)

    for core, acc, idx_hbm, out_hbm, n_out, n_pad in (
        (0, accp, cp_hbm, outp_hbm, np_out, np_pad),
        (1, accc, pc_hbm, outc_hbm, nc_out, nc_pad),
    ):
      r0 = n_pad // N_SUB
      w0 = n_out // N_SUB

      @pl.when(c == core)
      def _():
        def zero_step(k, _):
          pltpu.sync_copy(zbuf, acc.at[pl.ds(s * r0 + k * ZCH, ZCH)])
          return 0
        lax.fori_loop(0, r0 // ZCH, zero_step, 0)
        plsc.subcore_barrier()

        def edge_step(g, _):
          grow = s * (EPAD_SUB // GROUP) + g * GK
          pltpu.sync_copy(idx_hbm.at[pl.ds(grow, GK)], didx)
          for j in range(GK):
            pltpu.sync_copy(ones, acc.at[didx.at[j]], add=True)
          return 0
        lax.fori_loop(0, EPAD_SUB // GROUP // GK, edge_step, 0)
        plsc.subcore_barrier()
        pltpu.sync_copy(acc.at[pl.ds(s * w0, w0)], out_hbm.at[pl.ds(s * w0, w0)])

  return run(cpg, pcg)


def kernel(x_country, x_product, edge_src, edge_dst, Wc, bc, Wp, bp,
           W1_cp_l, b1_cp, W1_cp_r, W1_pc_l, b1_pc, W1_pc_r,
           W2_cp_l, b2_cp, W2_cp_r, W2_pc_l, b2_pc, W2_pc_r,
           Wih_c, Whh_c, bih_c, bhh_c, Wih_p, Whh_p, bih_p, bhh_p):
  es = edge_src.astype(i32)
  ed = edge_dst.astype(i32)
  pad = E_PAD - E

  def _g(a, fill):
    return jnp.concatenate([a, jnp.full((pad,), fill, i32)]).reshape(
        N_GROUPS, GROUP)

  cp_src = _g(es, 0)
  cp_dst = _g(ed, NP_N)
  pc_src = _g(ed, 0)
  pc_dst = _g(es, NC_N)
  eg_cp = jnp.stack([cp_src, cp_dst], axis=1)
  eg_pc = jnp.stack([pc_src, pc_dst], axis=1)

  r1 = lambda b: b.reshape(1, -1)

  cnt_p, cnt_c = _counts_sc(cp_dst, pc_dst)

  hc = _encode(x_country, Wc, r1(bc), NC_N, DC, 2000, slice_major=True)
  hp = _encode(x_product, Wp, r1(bp), NP_N, DP, 2000, slice_major=False)

  seg_p1 = _spmm_sc(hc, eg_cp, NC_N, NP_N, 32)
  seg_c1 = _spmm_sc(hp, eg_pc, NP_N, NC_N, 128)

  p1 = _combine(seg_p1, cnt_p, hp, W1_cp_l, r1(b1_cp), W1_cp_r,
                NP_N, 2000, relu=True, out_sl=False)
  c1 = _combine(seg_c1, cnt_c, hc, W1_pc_l, r1(b1_pc), W1_pc_r,
                NC_N, 2000, relu=True, out_sl=True)

  seg_p2 = _spmm_sc(c1, eg_cp, NC_N, NP_N, 32)
  seg_c2 = _spmm_sc(p1, eg_pc, NP_N, NC_N, 128)

  p2 = _combine(seg_p2, cnt_p, p1, W2_cp_l, r1(b2_cp), W2_cp_r,
                NP_N, 2000, relu=False, out_sl=False)
  c2 = _combine(seg_c2, cnt_c, c1, W2_pc_l, r1(b2_pc), W2_pc_r,
                NC_N, 2000, relu=False, out_sl=False)

  zc = _gru(c2, Wih_c, Whh_c, r1(bih_c), r1(bhh_c), NC_N, 2000)
  zp = _gru(p2, Wih_p, Whh_p, r1(bih_p), r1(bhh_p), NP_N, 2000)
  return (zc, zp)
